# Initial kernel scaffold; baseline (speedup 1.0000x reference)
import jax
import jax.numpy as jnp
from jax.experimental import pallas as pl


def kernel(edge_index, all_edge, e1, rel, entity_id, emb_e, emb_rel, gc2_W, gc2_b, alpha2, gc3_W, gc3_b, alpha3, bn3_g, bn3_b, bn4_g, bn4_b, dec_bias):
    raise NotImplementedError("write your pallas kernel here")


# speedup vs baseline: 1.9657x; 1.9657x over previous
import functools

import jax
import jax.numpy as jnp
from jax import lax
from jax.experimental import pallas as pl
from jax.experimental.pallas import tpu as pltpu
from jax.experimental.pallas import tpu_sc as plsc

N_ENT = 50000
N_REL = 16
D = 64
B = 1024
E = 800000

NC = 2
NS = 16
LANES = 16

HALF = 25088
NPAD = 2 * HALF
ACC_ROWS = 25216
TRASH = HALF
E_PAD = 819200
CH = 512
D_H = 32
N_CHUNK = E_PAD // NS // CH


def _sc_scatter_body(feats_hbm, src_hbm, dst_hbm, alp_hbm, out_hbm,
                     src_v, lidx_v, alp_v, rows_v, acc, sem):
    c = lax.axis_index("c")
    s = lax.axis_index("s")
    base_node = c * HALF

    zero16 = jnp.zeros((LANES,), jnp.float32)

    def zrow(i, _):
        for j in range(D_H // LANES):
            rows_v[i, pl.ds(j * LANES, LANES)] = zero16
        return 0

    lax.fori_loop(0, CH, zrow, 0)
    for off, sz in ((0, CH), (CH, CH), (2 * CH, CH), (3 * CH, 40)):
        pltpu.sync_copy(rows_v.at[pl.ds(0, sz)],
                        acc.at[pl.ds(s * 1576 + off, sz)])
    plsc.subcore_barrier()

    tile_base = s * (E_PAD // NS)

    def chunk(ci, _):
        eb = tile_base + ci * CH
        pltpu.sync_copy(src_hbm.at[pl.ds(eb, CH)], src_v)
        pltpu.sync_copy(alp_hbm.at[pl.ds(eb, CH)], alp_v)
        pltpu.sync_copy(dst_hbm.at[pl.ds(eb, CH)], lidx_v)
        pltpu.async_copy(feats_hbm.at[src_v], rows_v, sem).wait()

        def grp(g, _):
            a16 = alp_v[pl.ds(g * LANES, LANES)]
            for l in range(LANES):
                af = jnp.full((LANES,), a16[l], jnp.float32)
                row = g * LANES + l
                for j in range(D_H // LANES):
                    sl = pl.ds(j * LANES, LANES)
                    rows_v[row, sl] = rows_v[row, sl] * af
            d16 = lidx_v[pl.ds(g * LANES, LANES)]
            l16 = d16 - base_node
            ok = (l16 >= 0) & (l16 < HALF)
            l16 = jnp.where(ok, l16, TRASH)
            lidx_v[pl.ds(g * LANES, LANES)] = l16
            return 0

        lax.fori_loop(0, CH // LANES, grp, 0)
        pltpu.sync_copy(rows_v, acc.at[lidx_v], add=True)
        return 0

    lax.fori_loop(0, N_CHUNK, chunk, 0)
    plsc.subcore_barrier()

    pltpu.sync_copy(acc.at[pl.ds(s * 1568, 1568)],
                    out_hbm.at[pl.ds(base_node + s * 1568, 1568)])


def _sc_scatter(feats, src_pad, dst_pad, alp_pad):
    k = pl.kernel(
        _sc_scatter_body,
        out_type=jax.ShapeDtypeStruct((NPAD, D_H), jnp.float32),
        mesh=plsc.VectorSubcoreMesh(core_axis_name="c", subcore_axis_name="s"),
        scratch_types=[
            pltpu.VMEM((CH,), jnp.int32),
            pltpu.VMEM((CH,), jnp.int32),
            pltpu.VMEM((CH,), jnp.float32),
            pltpu.VMEM((CH, D_H), jnp.float32),
            pltpu.VMEM_SHARED((ACC_ROWS, D_H), jnp.float32),
            pltpu.SemaphoreType.DMA,
        ],
        compiler_params=pltpu.CompilerParams(use_tc_tiling_on_sc=False),
    )
    return k(feats, src_pad, dst_pad, alp_pad)


def _sc_obj_body(eall_hbm, rel_t_hbm, e1_hbm, rel_hbm, out_hbm,
                 e1_v, rel_v, erow_v, rrow_v, sem):
    c = lax.axis_index("c")
    s = lax.axis_index("s")
    wid = s * NC + c
    per = B // (NC * NS)
    base = wid * per
    pltpu.sync_copy(e1_hbm.at[pl.ds(base, per)], e1_v)
    pltpu.sync_copy(rel_hbm.at[pl.ds(base, per)], rel_v)
    pltpu.async_copy(eall_hbm.at[e1_v], erow_v, sem).wait()
    pltpu.async_copy(rel_t_hbm.at[rel_v], rrow_v, sem).wait()
    for i in range(B // (NC * NS)):
        for j in range(D // LANES):
            sl = pl.ds(j * LANES, LANES)
            erow_v[i, sl] = erow_v[i, sl] * rrow_v[i, sl]
    pltpu.sync_copy(erow_v, out_hbm.at[pl.ds(base, per)])


def _sc_obj(eall, emb_rel, e1_flat, rel_flat):
    per = B // (NC * NS)
    k = pl.kernel(
        _sc_obj_body,
        out_type=jax.ShapeDtypeStruct((B, D), jnp.float32),
        mesh=plsc.VectorSubcoreMesh(core_axis_name="c", subcore_axis_name="s"),
        scratch_types=[
            pltpu.VMEM((per,), jnp.int32),
            pltpu.VMEM((per,), jnp.int32),
            pltpu.VMEM((per, D), jnp.float32),
            pltpu.VMEM((per, D), jnp.float32),
            pltpu.SemaphoreType.DMA,
        ],
        compiler_params=pltpu.CompilerParams(use_tc_tiling_on_sc=False),
    )
    return k(eall, emb_rel, e1_flat, rel_flat)


def _alp_body(et_ref, ett_ref, a2_ref, a3_ref, o2_ref, o3_ref):
    et = et_ref[...]
    ett = ett_ref[...]
    acc2 = jnp.zeros(et.shape, jnp.float32)
    acc3 = jnp.zeros(et.shape, jnp.float32)
    for t in range(N_REL + 1):
        ind = (et == t).astype(jnp.float32) + (ett == t).astype(jnp.float32)
        acc2 = acc2 + a2_ref[0, t] * ind
        acc3 = acc3 + a3_ref[0, t] * ind
    o2_ref[...] = acc2
    o3_ref[...] = acc3


def _alp(et2d, ett2d, a2row, a3row):
    rows = et2d.shape[0]
    blk = rows // 5
    return pl.pallas_call(
        _alp_body,
        grid=(5,),
        in_specs=[
            pl.BlockSpec((blk, 128), lambda i: (i, 0)),
            pl.BlockSpec((blk, 128), lambda i: (i, 0)),
            pl.BlockSpec((1, 128), lambda i: (0, 0)),
            pl.BlockSpec((1, 128), lambda i: (0, 0)),
        ],
        out_specs=[
            pl.BlockSpec((blk, 128), lambda i: (i, 0)),
            pl.BlockSpec((blk, 128), lambda i: (i, 0)),
        ],
        out_shape=[
            jax.ShapeDtypeStruct(et2d.shape, jnp.float32),
            jax.ShapeDtypeStruct(et2d.shape, jnp.float32),
        ],
    )(et2d, ett2d, a2row, a3row)


def _mm_body(x_ref, w_ref, o_ref):
    o_ref[...] = jnp.dot(x_ref[...], w_ref[...],
                         preferred_element_type=jnp.float32)


def _mm(x, w):
    n = x.shape[0]
    blk = 2000
    return pl.pallas_call(
        _mm_body,
        grid=(n // blk,),
        in_specs=[
            pl.BlockSpec((blk, D), lambda i: (i, 0)),
            pl.BlockSpec((D, D), lambda i: (0, 0)),
        ],
        out_specs=pl.BlockSpec((blk, D), lambda i: (i, 0)),
        out_shape=jax.ShapeDtypeStruct((n, D), jnp.float32),
    )(x, w)


def _stats_body(x_ref, o_ref):
    i = pl.program_id(0)

    @pl.when(i == 0)
    def _():
        o_ref[...] = jnp.zeros_like(o_ref)

    x = x_ref[...]
    o_ref[0:1, :] = o_ref[0:1, :] + jnp.sum(x, axis=0, keepdims=True)
    o_ref[1:2, :] = o_ref[1:2, :] + jnp.sum(x * x, axis=0, keepdims=True)


def _stats(x):
    n = x.shape[0]
    blk = 2000
    return pl.pallas_call(
        _stats_body,
        grid=(n // blk,),
        in_specs=[pl.BlockSpec((blk, D), lambda i: (i, 0))],
        out_specs=pl.BlockSpec((8, D), lambda i: (0, 0)),
        out_shape=jax.ShapeDtypeStruct((8, D), jnp.float32),
    )(x)


def _bn_tanh_mm_body(x_ref, st_ref, g_ref, b_ref, w_ref, o_ref, *, n, with_mm):
    mu = st_ref[0:1, :] / n
    ex2 = st_ref[1:2, :] / n
    var = ex2 - mu * mu
    inv = g_ref[...] * lax.rsqrt(var + 1e-5)
    h = jnp.tanh((x_ref[...] - mu) * inv + b_ref[...])
    if with_mm:
        o_ref[...] = jnp.dot(h, w_ref[...], preferred_element_type=jnp.float32)
    else:
        o_ref[...] = h


def _bn_tanh(x, st, g2d, b2d, w, with_mm):
    n = x.shape[0]
    blk = 2000
    return pl.pallas_call(
        functools.partial(_bn_tanh_mm_body, n=float(n), with_mm=with_mm),
        grid=(n // blk,),
        in_specs=[
            pl.BlockSpec((blk, D), lambda i: (i, 0)),
            pl.BlockSpec((8, D), lambda i: (0, 0)),
            pl.BlockSpec((1, D), lambda i: (0, 0)),
            pl.BlockSpec((1, D), lambda i: (0, 0)),
            pl.BlockSpec((D, D), lambda i: (0, 0)),
        ],
        out_specs=pl.BlockSpec((blk, D), lambda i: (i, 0)),
        out_shape=jax.ShapeDtypeStruct((n, D), jnp.float32),
    )(x, st, g2d, b2d, w)


def _dec_body(obj_ref, e_ref, bias_ref, o_ref):
    l = lax.dot_general(obj_ref[...], e_ref[...],
                        (((1,), (1,)), ((), ())),
                        preferred_element_type=jnp.float32)
    o_ref[...] = jax.nn.sigmoid(l + bias_ref[...])


def _decoder(obj, eall_pad, bias2d):
    npad = eall_pad.shape[0]
    blk = 1024
    return pl.pallas_call(
        _dec_body,
        grid=(npad // blk,),
        in_specs=[
            pl.BlockSpec((B, D), lambda i: (0, 0)),
            pl.BlockSpec((blk, D), lambda i: (i, 0)),
            pl.BlockSpec((1, blk), lambda i: (0, i)),
        ],
        out_specs=pl.BlockSpec((B, blk), lambda i: (0, i)),
        out_shape=jax.ShapeDtypeStruct((B, npad), jnp.float32),
    )(obj, eall_pad, bias2d)


def kernel(edge_index, all_edge, e1, rel, entity_id, emb_e, emb_rel,
           gc2_W, gc2_b, alpha2, gc3_W, gc3_b, alpha3,
           bn3_g, bn3_b, bn4_g, bn4_b, dec_bias):
    n = entity_id.shape[0]
    T = (all_edge.shape[0] - n) // 2

    src = edge_index[0].astype(jnp.int32)
    dst = edge_index[1].astype(jnp.int32)
    et = all_edge.astype(jnp.int32)
    ett = jnp.concatenate([et[T:2 * T], et[:T], et[-n:]])

    pad_e = E_PAD - E
    src_pad = jnp.concatenate([src, jnp.zeros((pad_e,), jnp.int32)])
    dst_pad = jnp.concatenate([dst, jnp.full((pad_e,), 2 * NPAD, jnp.int32)])
    et2d = jnp.concatenate([et, jnp.zeros((pad_e,), jnp.int32)]) \
        .reshape(E_PAD // 128, 128)
    ett2d = jnp.concatenate([ett, jnp.zeros((pad_e,), jnp.int32)]) \
        .reshape(E_PAD // 128, 128)
    a2row = jnp.pad(alpha2[:, 0], (0, 128 - (N_REL + 1))).reshape(1, 128)
    a3row = jnp.pad(alpha3[:, 0], (0, 128 - (N_REL + 1))).reshape(1, 128)
    alp2_2d, alp3_2d = _alp(et2d, ett2d, a2row, a3row)
    alp2_pad = alp2_2d.reshape(-1)
    alp3_pad = alp3_2d.reshape(-1)

    g3 = bn3_g.reshape(1, D)
    b3 = bn3_b.reshape(1, D)
    g4 = bn4_g.reshape(1, D)
    b4 = bn4_b.reshape(1, D)

    def scatter64(feats, alp_pad):
        lo = _sc_scatter(feats[:, :D_H] * 1.0,
                         src_pad, dst_pad, alp_pad)
        hi = _sc_scatter(feats[:, D_H:] * 1.0,
                         src_pad, dst_pad, alp_pad)
        return jnp.concatenate([lo[:n], hi[:n]], axis=1)

    feats1 = _mm(emb_e, gc2_W)
    y1 = scatter64(feats1, alp2_pad)
    st1 = _stats(y1)
    feats2 = _bn_tanh(y1, st1, g3, b3, gc3_W, True)

    y2 = scatter64(feats2, alp3_pad)
    st2 = _stats(y2)
    eall = _bn_tanh(y2, st2, g4, b4, gc3_W, False)

    obj = _sc_obj(eall, emb_rel, e1[:, 0].astype(jnp.int32),
                  rel[:, 0].astype(jnp.int32))
    npad = 50176
    eall_pad = jnp.pad(eall, ((0, npad - n), (0, 0)))
    bias2d = jnp.pad(dec_bias, (0, npad - n)).reshape(1, npad)
    logits = _decoder(obj, eall_pad, bias2d)
    return logits[:, :n]


# speedup vs baseline: 2.0773x; 1.0568x over previous
import functools

import jax
import jax.numpy as jnp
from jax import lax
from jax.experimental import pallas as pl
from jax.experimental.pallas import tpu as pltpu
from jax.experimental.pallas import tpu_sc as plsc

N_ENT = 50000
N_REL = 16
D = 64
B = 1024
E = 800000

NC = 2
NS = 16
LANES = 16

HALF = 25088
NPAD = 2 * HALF
ACC_ROWS = 25216
TRASH = HALF
E_PAD = 819200
CH = 512
D_H = 32
N_CHUNK = E_PAD // NS // CH


def _sc_scatter_body(feats_hbm, src_hbm, dst_hbm, alp_hbm, out_hbm,
                     src_v, src_v2, lidx_v, alp_v, rows_v, rows_v2,
                     acc, sem, sem2):
    srcs = (src_v, src_v2)
    rows = (rows_v, rows_v2)
    sems = (sem, sem2)
    c = lax.axis_index("c")
    s = lax.axis_index("s")
    base_node = c * HALF

    zero16 = jnp.zeros((LANES,), jnp.float32)

    def zrow(i, _):
        for j in range(D_H // LANES):
            rows_v[i, pl.ds(j * LANES, LANES)] = zero16
        return 0

    lax.fori_loop(0, CH, zrow, 0)
    for off, sz in ((0, CH), (CH, CH), (2 * CH, CH), (3 * CH, 40)):
        pltpu.sync_copy(rows_v.at[pl.ds(0, sz)],
                        acc.at[pl.ds(s * 1576 + off, sz)])
    plsc.subcore_barrier()

    tile_base = s * (E_PAD // NS)

    pltpu.sync_copy(src_hbm.at[pl.ds(tile_base, CH)], src_v)
    pltpu.async_copy(feats_hbm.at[src_v], rows_v, sem)

    def pair(i, _):
        for b in range(2):
            ci = 2 * i + b
            rv = rows[b]
            pltpu.make_async_copy(feats_hbm.at[srcs[b]], rv, sems[b]).wait()

            @pl.when(ci + 1 < N_CHUNK)
            def _():
                nb = tile_base + (ci + 1) * CH
                pltpu.sync_copy(src_hbm.at[pl.ds(nb, CH)], srcs[1 - b])
                pltpu.async_copy(feats_hbm.at[srcs[1 - b]], rows[1 - b],
                                 sems[1 - b])

            eb = tile_base + ci * CH
            pltpu.sync_copy(alp_hbm.at[pl.ds(eb, CH)], alp_v)
            pltpu.sync_copy(dst_hbm.at[pl.ds(eb, CH)], lidx_v)

            def grp(g, _):
                a16 = alp_v[pl.ds(g * LANES, LANES)]
                for l in range(LANES):
                    af = jnp.full((LANES,), a16[l], jnp.float32)
                    row = g * LANES + l
                    for j in range(D_H // LANES):
                        sl = pl.ds(j * LANES, LANES)
                        rv[row, sl] = rv[row, sl] * af
                d16 = lidx_v[pl.ds(g * LANES, LANES)]
                l16 = d16 - base_node
                ok = (l16 >= 0) & (l16 < HALF)
                l16 = jnp.where(ok, l16, TRASH)
                lidx_v[pl.ds(g * LANES, LANES)] = l16
                return 0

            lax.fori_loop(0, CH // LANES, grp, 0)
            pltpu.sync_copy(rv, acc.at[lidx_v], add=True)
        return 0

    lax.fori_loop(0, N_CHUNK // 2, pair, 0)
    plsc.subcore_barrier()

    pltpu.sync_copy(acc.at[pl.ds(s * 1568, 1568)],
                    out_hbm.at[pl.ds(base_node + s * 1568, 1568)])


def _sc_scatter(feats, src_pad, dst_pad, alp_pad):
    k = pl.kernel(
        _sc_scatter_body,
        out_type=jax.ShapeDtypeStruct((NPAD, D_H), jnp.float32),
        mesh=plsc.VectorSubcoreMesh(core_axis_name="c", subcore_axis_name="s"),
        scratch_types=[
            pltpu.VMEM((CH,), jnp.int32),
            pltpu.VMEM((CH,), jnp.int32),
            pltpu.VMEM((CH,), jnp.int32),
            pltpu.VMEM((CH,), jnp.float32),
            pltpu.VMEM((CH, D_H), jnp.float32),
            pltpu.VMEM((CH, D_H), jnp.float32),
            pltpu.VMEM_SHARED((ACC_ROWS, D_H), jnp.float32),
            pltpu.SemaphoreType.DMA,
            pltpu.SemaphoreType.DMA,
        ],
        compiler_params=pltpu.CompilerParams(use_tc_tiling_on_sc=False),
    )
    return k(feats, src_pad, dst_pad, alp_pad)


def _sc_obj_body(eall_hbm, rel_t_hbm, e1_hbm, rel_hbm, out_hbm,
                 e1_v, rel_v, erow_v, rrow_v, sem):
    c = lax.axis_index("c")
    s = lax.axis_index("s")
    wid = s * NC + c
    per = B // (NC * NS)
    base = wid * per
    pltpu.sync_copy(e1_hbm.at[pl.ds(base, per)], e1_v)
    pltpu.sync_copy(rel_hbm.at[pl.ds(base, per)], rel_v)
    pltpu.async_copy(eall_hbm.at[e1_v], erow_v, sem).wait()
    pltpu.async_copy(rel_t_hbm.at[rel_v], rrow_v, sem).wait()
    for i in range(B // (NC * NS)):
        for j in range(D // LANES):
            sl = pl.ds(j * LANES, LANES)
            erow_v[i, sl] = erow_v[i, sl] * rrow_v[i, sl]
    pltpu.sync_copy(erow_v, out_hbm.at[pl.ds(base, per)])


def _sc_obj(eall, emb_rel, e1_flat, rel_flat):
    per = B // (NC * NS)
    k = pl.kernel(
        _sc_obj_body,
        out_type=jax.ShapeDtypeStruct((B, D), jnp.float32),
        mesh=plsc.VectorSubcoreMesh(core_axis_name="c", subcore_axis_name="s"),
        scratch_types=[
            pltpu.VMEM((per,), jnp.int32),
            pltpu.VMEM((per,), jnp.int32),
            pltpu.VMEM((per, D), jnp.float32),
            pltpu.VMEM((per, D), jnp.float32),
            pltpu.SemaphoreType.DMA,
        ],
        compiler_params=pltpu.CompilerParams(use_tc_tiling_on_sc=False),
    )
    return k(eall, emb_rel, e1_flat, rel_flat)


def _alp_body(et_ref, ett_ref, a2_ref, a3_ref, o2_ref, o3_ref):
    et = et_ref[...]
    ett = ett_ref[...]
    acc2 = jnp.zeros(et.shape, jnp.float32)
    acc3 = jnp.zeros(et.shape, jnp.float32)
    for t in range(N_REL + 1):
        ind = (et == t).astype(jnp.float32) + (ett == t).astype(jnp.float32)
        acc2 = acc2 + a2_ref[0, t] * ind
        acc3 = acc3 + a3_ref[0, t] * ind
    o2_ref[...] = acc2
    o3_ref[...] = acc3


def _alp(et2d, ett2d, a2row, a3row):
    rows = et2d.shape[0]
    blk = rows // 5
    return pl.pallas_call(
        _alp_body,
        grid=(5,),
        in_specs=[
            pl.BlockSpec((blk, 128), lambda i: (i, 0)),
            pl.BlockSpec((blk, 128), lambda i: (i, 0)),
            pl.BlockSpec((1, 128), lambda i: (0, 0)),
            pl.BlockSpec((1, 128), lambda i: (0, 0)),
        ],
        out_specs=[
            pl.BlockSpec((blk, 128), lambda i: (i, 0)),
            pl.BlockSpec((blk, 128), lambda i: (i, 0)),
        ],
        out_shape=[
            jax.ShapeDtypeStruct(et2d.shape, jnp.float32),
            jax.ShapeDtypeStruct(et2d.shape, jnp.float32),
        ],
    )(et2d, ett2d, a2row, a3row)


def _mm_body(x_ref, w_ref, o_ref):
    o_ref[...] = jnp.dot(x_ref[...], w_ref[...],
                         preferred_element_type=jnp.float32)


def _mm(x, w):
    n = x.shape[0]
    blk = 2000
    return pl.pallas_call(
        _mm_body,
        grid=(n // blk,),
        in_specs=[
            pl.BlockSpec((blk, D), lambda i: (i, 0)),
            pl.BlockSpec((D, D), lambda i: (0, 0)),
        ],
        out_specs=pl.BlockSpec((blk, D), lambda i: (i, 0)),
        out_shape=jax.ShapeDtypeStruct((n, D), jnp.float32),
    )(x, w)


def _stats_body(x_ref, o_ref):
    i = pl.program_id(0)

    @pl.when(i == 0)
    def _():
        o_ref[...] = jnp.zeros_like(o_ref)

    x = x_ref[...]
    o_ref[0:1, :] = o_ref[0:1, :] + jnp.sum(x, axis=0, keepdims=True)
    o_ref[1:2, :] = o_ref[1:2, :] + jnp.sum(x * x, axis=0, keepdims=True)


def _stats(x):
    n = x.shape[0]
    blk = 2000
    return pl.pallas_call(
        _stats_body,
        grid=(n // blk,),
        in_specs=[pl.BlockSpec((blk, D), lambda i: (i, 0))],
        out_specs=pl.BlockSpec((8, D), lambda i: (0, 0)),
        out_shape=jax.ShapeDtypeStruct((8, D), jnp.float32),
    )(x)


def _bn_tanh_mm_body(x_ref, st_ref, g_ref, b_ref, w_ref, o_ref, *, n, with_mm):
    mu = st_ref[0:1, :] / n
    ex2 = st_ref[1:2, :] / n
    var = ex2 - mu * mu
    inv = g_ref[...] * lax.rsqrt(var + 1e-5)
    h = jnp.tanh((x_ref[...] - mu) * inv + b_ref[...])
    if with_mm:
        o_ref[...] = jnp.dot(h, w_ref[...], preferred_element_type=jnp.float32)
    else:
        o_ref[...] = h


def _bn_tanh(x, st, g2d, b2d, w, with_mm):
    n = x.shape[0]
    blk = 2000
    return pl.pallas_call(
        functools.partial(_bn_tanh_mm_body, n=float(n), with_mm=with_mm),
        grid=(n // blk,),
        in_specs=[
            pl.BlockSpec((blk, D), lambda i: (i, 0)),
            pl.BlockSpec((8, D), lambda i: (0, 0)),
            pl.BlockSpec((1, D), lambda i: (0, 0)),
            pl.BlockSpec((1, D), lambda i: (0, 0)),
            pl.BlockSpec((D, D), lambda i: (0, 0)),
        ],
        out_specs=pl.BlockSpec((blk, D), lambda i: (i, 0)),
        out_shape=jax.ShapeDtypeStruct((n, D), jnp.float32),
    )(x, st, g2d, b2d, w)


def _dec_body(obj_ref, e_ref, bias_ref, o_ref):
    l = lax.dot_general(obj_ref[...], e_ref[...],
                        (((1,), (1,)), ((), ())),
                        preferred_element_type=jnp.float32)
    o_ref[...] = jax.nn.sigmoid(l + bias_ref[...])


def _decoder(obj, eall_pad, bias2d):
    npad = eall_pad.shape[0]
    blk = 1024
    return pl.pallas_call(
        _dec_body,
        grid=(npad // blk,),
        in_specs=[
            pl.BlockSpec((B, D), lambda i: (0, 0)),
            pl.BlockSpec((blk, D), lambda i: (i, 0)),
            pl.BlockSpec((1, blk), lambda i: (0, i)),
        ],
        out_specs=pl.BlockSpec((B, blk), lambda i: (0, i)),
        out_shape=jax.ShapeDtypeStruct((B, npad), jnp.float32),
    )(obj, eall_pad, bias2d)


def kernel(edge_index, all_edge, e1, rel, entity_id, emb_e, emb_rel,
           gc2_W, gc2_b, alpha2, gc3_W, gc3_b, alpha3,
           bn3_g, bn3_b, bn4_g, bn4_b, dec_bias):
    n = entity_id.shape[0]
    T = (all_edge.shape[0] - n) // 2

    src = edge_index[0].astype(jnp.int32)
    dst = edge_index[1].astype(jnp.int32)
    et = all_edge.astype(jnp.int32)
    ett = jnp.concatenate([et[T:2 * T], et[:T], et[-n:]])

    pad_e = E_PAD - E
    src_pad = jnp.concatenate([src, jnp.zeros((pad_e,), jnp.int32)])
    dst_pad = jnp.concatenate([dst, jnp.full((pad_e,), 2 * NPAD, jnp.int32)])
    et2d = jnp.concatenate([et, jnp.zeros((pad_e,), jnp.int32)]) \
        .reshape(E_PAD // 128, 128)
    ett2d = jnp.concatenate([ett, jnp.zeros((pad_e,), jnp.int32)]) \
        .reshape(E_PAD // 128, 128)
    a2row = jnp.pad(alpha2[:, 0], (0, 128 - (N_REL + 1))).reshape(1, 128)
    a3row = jnp.pad(alpha3[:, 0], (0, 128 - (N_REL + 1))).reshape(1, 128)
    alp2_2d, alp3_2d = _alp(et2d, ett2d, a2row, a3row)
    alp2_pad = alp2_2d.reshape(-1)
    alp3_pad = alp3_2d.reshape(-1)

    g3 = bn3_g.reshape(1, D)
    b3 = bn3_b.reshape(1, D)
    g4 = bn4_g.reshape(1, D)
    b4 = bn4_b.reshape(1, D)

    def scatter64(feats, alp_pad):
        lo = _sc_scatter(feats[:, :D_H] * 1.0,
                         src_pad, dst_pad, alp_pad)
        hi = _sc_scatter(feats[:, D_H:] * 1.0,
                         src_pad, dst_pad, alp_pad)
        return jnp.concatenate([lo[:n], hi[:n]], axis=1)

    feats1 = _mm(emb_e, gc2_W)
    y1 = scatter64(feats1, alp2_pad)
    st1 = _stats(y1)
    feats2 = _bn_tanh(y1, st1, g3, b3, gc3_W, True)

    y2 = scatter64(feats2, alp3_pad)
    st2 = _stats(y2)
    eall = _bn_tanh(y2, st2, g4, b4, gc3_W, False)

    obj = _sc_obj(eall, emb_rel, e1[:, 0].astype(jnp.int32),
                  rel[:, 0].astype(jnp.int32))
    npad = 50176
    eall_pad = jnp.pad(eall, ((0, npad - n), (0, 0)))
    bias2d = jnp.pad(dec_bias, (0, npad - n)).reshape(1, npad)
    logits = _decoder(obj, eall_pad, bias2d)
    return logits[:, :n]
